# baseline (speedup 1.0000x reference)
import functools

import jax
import jax.numpy as jnp
from jax import lax
from jax.experimental import pallas as pl
from jax.experimental.pallas import tpu as pltpu

_LANE = 128
_TM1 = 512
_TM2 = 1024
_VMEM = 64 * 1024 * 1024
_QSCALE = 128.0


def _ceil_to(v, m):
    return ((v + m - 1) // m) * m


def _agg1_kernel(adj_ref, x_ref, wp_ref, o_ref, adjq_ref, *,
                 nfeat, nhid, f23):
    a = adj_ref[...]
    adjq_ref[...] = (a * _QSCALE).astype(adjq_ref.dtype)
    t = jnp.dot(
        a.astype(jnp.bfloat16), x_ref[...].astype(jnp.bfloat16),
        preferred_element_type=jnp.float32,
    )
    w1 = wp_ref[0:nfeat, 0:nhid]
    b1 = wp_ref[nfeat:nfeat + 1, 0:nhid]
    w23 = wp_ref[nfeat + 8:nfeat + 8 + nhid, 0:f23].astype(jnp.bfloat16)
    h = jnp.dot(t, w1, preferred_element_type=jnp.float32)
    h = jnp.maximum(h + b1, 0.0).astype(jnp.bfloat16)
    o_ref[...] = jnp.dot(
        h, w23, preferred_element_type=jnp.float32
    ).astype(o_ref.dtype)


def _agg2_kernel(adjq_ref, sup_ref, wp_ref, o_ref, *, b23_row, c1, c2):
    z = jnp.dot(adjq_ref[...], sup_ref[...], preferred_element_type=jnp.float32)
    f23 = z.shape[1]
    b23 = wp_ref[b23_row:b23_row + 1, 0:f23]
    z = z * (1.0 / _QSCALE) + b23

    lane = lax.broadcasted_iota(jnp.int32, z.shape, 1)
    minus_inf = jnp.float32(-jnp.inf)

    def group_stats(mask):
        zg = jnp.where(mask, z, minus_inf)
        m = jnp.max(zg, axis=-1, keepdims=True)
        e = jnp.exp(zg - m)
        return m, e, jnp.sum(e, axis=-1, keepdims=True)

    mask1 = lane < c1
    mask2 = (lane >= c1) & (lane < c1 + c2)
    m1, e1, s1 = group_stats(mask1)
    m2, _, s2 = group_stats(mask2)
    prob_last = (
        jnp.sum(jnp.where(lane == c1 - 1, e1, 0.0), axis=-1, keepdims=True)
        / s1
    )
    out = jnp.where(
        mask1,
        z - m1 - jnp.log(s1),
        jnp.where(mask2, z - m2 - jnp.log(s2), prob_last),
    )
    o_ref[...] = out[:, :o_ref.shape[1]]


def _row_slab_call(body, n_p, tm, out_shapes, out_specs, operands,
                   operand_specs):
    return pl.pallas_call(
        body,
        out_shape=out_shapes,
        grid=(n_p // tm,),
        in_specs=operand_specs,
        out_specs=out_specs,
        compiler_params=pltpu.CompilerParams(
            dimension_semantics=("parallel",),
            vmem_limit_bytes=_VMEM,
        ),
    )(*operands)


def kernel(gc1_w, gc1_b, gc2_w, gc2_b, gc3_w, gc3_b, x, adj):
    n, nfeat = x.shape
    nhid = gc1_w.shape[1]
    c1 = gc2_w.shape[1]
    c2 = gc3_w.shape[1]
    f23 = _ceil_to(c1 + c2 + 1, _LANE)

    n_p = _ceil_to(n, max(_TM1, _TM2))
    if n_p != n:
        x = jnp.pad(x, ((0, n_p - n), (0, 0)))
        adj = jnp.pad(adj, ((0, n_p - n), (0, n_p - n)))

    f_out = 16

    wpw = max(nhid, f23)
    b23_row = nfeat + 8 + nhid
    wp = jnp.concatenate([
        gc1_w,
        gc1_b.reshape(1, nhid),
        jnp.zeros((7, wpw), jnp.float32),
        jnp.pad(jnp.concatenate([gc2_w, gc3_w], axis=1),
                ((0, 0), (0, wpw - c1 - c2))),
        jnp.pad(jnp.concatenate([gc2_b, gc3_b]).reshape(1, c1 + c2),
                ((0, 0), (0, wpw - c1 - c2))),
        jnp.zeros((7, wpw), jnp.float32),
    ], axis=0)

    whole = lambda shape: pl.BlockSpec(shape, lambda i: (0,) * len(shape))
    row_slab = lambda tm, cols: pl.BlockSpec((tm, cols), lambda i: (i, 0))
    f8 = jnp.float8_e4m3fn

    sup23, adjq = _row_slab_call(
        functools.partial(_agg1_kernel, nfeat=nfeat, nhid=nhid, f23=f23),
        n_p, _TM1,
        (jax.ShapeDtypeStruct((n_p, f23), f8),
         jax.ShapeDtypeStruct((n_p, n_p), f8)),
        (row_slab(_TM1, f23), row_slab(_TM1, n_p)),
        (adj, x, wp),
        [row_slab(_TM1, n_p), whole((n_p, nfeat)), whole(wp.shape)],
    )
    slab = _row_slab_call(
        functools.partial(_agg2_kernel, b23_row=b23_row, c1=c1, c2=c2),
        n_p, _TM2,
        jax.ShapeDtypeStruct((n_p, f_out), jnp.float32),
        row_slab(_TM2, f_out),
        (adjq, sup23, wp),
        [row_slab(_TM2, n_p), whole((n_p, f23)), whole(wp.shape)],
    )

    return slab[:n, :c1], slab[:n, c1:c1 + c2], slab[:n, c1 + c2]


# speedup vs baseline: 1.0571x; 1.0571x over previous
import functools

import jax
import jax.numpy as jnp
from jax import lax
from jax.experimental import pallas as pl
from jax.experimental.pallas import tpu as pltpu

_LANE = 128
_TM1 = 512
_TM2 = 1024
_VMEM = 64 * 1024 * 1024
_QSCALE = 128.0


def _ceil_to(v, m):
    return ((v + m - 1) // m) * m


def _agg1_kernel(adj_ref, x_ref, w1_ref, b1_ref, w2_ref, w3_ref, o_ref,
                 adjq_ref, *, f23):
    a = adj_ref[...]
    adjq_ref[...] = (a * _QSCALE).astype(adjq_ref.dtype)
    t = jnp.dot(
        a.astype(jnp.bfloat16), x_ref[...].astype(jnp.bfloat16),
        preferred_element_type=jnp.float32,
    )
    h = jnp.dot(t, w1_ref[...], preferred_element_type=jnp.float32)
    h = jnp.maximum(h + b1_ref[...][None, :], 0.0).astype(jnp.bfloat16)
    w2, w3 = w2_ref[...], w3_ref[...]
    pad = jnp.zeros(
        (w2.shape[0], f23 - w2.shape[1] - w3.shape[1]), jnp.float32)
    w23 = jnp.concatenate([w2, w3, pad], axis=1).astype(jnp.bfloat16)
    o_ref[...] = jnp.dot(
        h, w23, preferred_element_type=jnp.float32
    ).astype(o_ref.dtype)


def _agg2_kernel(adjq_ref, sup_ref, b2_ref, b3_ref, o_ref, *, c1, c2):
    z = jnp.dot(adjq_ref[...], sup_ref[...], preferred_element_type=jnp.float32)
    f23 = z.shape[1]
    pad = jnp.zeros((f23 - c1 - c2,), jnp.float32)
    b23 = jnp.concatenate([b2_ref[...], b3_ref[...], pad])
    z = z * (1.0 / _QSCALE) + b23[None, :]

    lane = lax.broadcasted_iota(jnp.int32, z.shape, 1)
    minus_inf = jnp.float32(-jnp.inf)

    def group_stats(mask):
        zg = jnp.where(mask, z, minus_inf)
        m = jnp.max(zg, axis=-1, keepdims=True)
        e = jnp.exp(zg - m)
        return m, e, jnp.sum(e, axis=-1, keepdims=True)

    mask1 = lane < c1
    mask2 = (lane >= c1) & (lane < c1 + c2)
    m1, e1, s1 = group_stats(mask1)
    m2, _, s2 = group_stats(mask2)
    prob_last = (
        jnp.sum(jnp.where(lane == c1 - 1, e1, 0.0), axis=-1, keepdims=True)
        / s1
    )
    out = jnp.where(
        mask1,
        z - m1 - jnp.log(s1),
        jnp.where(mask2, z - m2 - jnp.log(s2), prob_last),
    )
    o_ref[...] = out[:, :o_ref.shape[1]]


def _row_slab_call(body, n_p, tm, out_shapes, out_specs, operands,
                   operand_specs):
    return pl.pallas_call(
        body,
        out_shape=out_shapes,
        grid=(n_p // tm,),
        in_specs=operand_specs,
        out_specs=out_specs,
        compiler_params=pltpu.CompilerParams(
            dimension_semantics=("parallel",),
            vmem_limit_bytes=_VMEM,
        ),
    )(*operands)


def kernel(gc1_w, gc1_b, gc2_w, gc2_b, gc3_w, gc3_b, x, adj):
    n, nfeat = x.shape
    nhid = gc1_w.shape[1]
    c1 = gc2_w.shape[1]
    c2 = gc3_w.shape[1]
    f23 = _ceil_to(c1 + c2 + 1, _LANE)

    n_p = _ceil_to(n, max(_TM1, _TM2))
    if n_p != n:
        x = jnp.pad(x, ((0, n_p - n), (0, 0)))
        adj = jnp.pad(adj, ((0, n_p - n), (0, n_p - n)))

    f_out = 16

    whole = lambda shape: pl.BlockSpec(shape, lambda i: (0,) * len(shape))
    row_slab = lambda tm, cols: pl.BlockSpec((tm, cols), lambda i: (i, 0))
    f8 = jnp.float8_e4m3fn

    sup23, adjq = _row_slab_call(
        functools.partial(_agg1_kernel, f23=f23),
        n_p, _TM1,
        (jax.ShapeDtypeStruct((n_p, f23), f8),
         jax.ShapeDtypeStruct((n_p, n_p), f8)),
        (row_slab(_TM1, f23), row_slab(_TM1, n_p)),
        (adj, x, gc1_w, gc1_b, gc2_w, gc3_w),
        [row_slab(_TM1, n_p), whole((n_p, nfeat)), whole((nfeat, nhid)),
         whole((nhid,)), whole((nhid, c1)), whole((nhid, c2))],
    )
    slab = _row_slab_call(
        functools.partial(_agg2_kernel, c1=c1, c2=c2),
        n_p, _TM2,
        jax.ShapeDtypeStruct((n_p, f_out), jnp.float32),
        row_slab(_TM2, f_out),
        (adjq, sup23, gc2_b, gc3_b),
        [row_slab(_TM2, n_p), whole((n_p, f23)), whole((c1,)),
         whole((c2,))],
    )

    return slab[:n, :c1], slab[:n, c1:c1 + c2], slab[:n, c1 + c2]


# speedup vs baseline: 1.0623x; 1.0049x over previous
import functools

import jax
import jax.numpy as jnp
from jax import lax
from jax.experimental import pallas as pl
from jax.experimental.pallas import tpu as pltpu

_LANE = 128
_TM1 = 512
_TM2 = 1024
_VMEM = 64 * 1024 * 1024
_QSCALE = 128.0


def _ceil_to(v, m):
    return ((v + m - 1) // m) * m


def _agg1_kernel(adj_ref, x_ref, w1_ref, b1_ref, w2_ref, w3_ref, o_ref,
                 adjq_ref, *, f23):
    a = adj_ref[...]
    adjq_ref[...] = (a * _QSCALE).astype(adjq_ref.dtype)
    t = jnp.dot(
        a.astype(jnp.bfloat16), x_ref[...].astype(jnp.bfloat16),
        preferred_element_type=jnp.float32,
    )
    h = jnp.dot(t, w1_ref[...], preferred_element_type=jnp.float32)
    h = jnp.maximum(h + b1_ref[...][None, :], 0.0).astype(jnp.bfloat16)
    w2, w3 = w2_ref[...], w3_ref[...]
    pad = jnp.zeros(
        (w2.shape[0], f23 - w2.shape[1] - w3.shape[1]), jnp.float32)
    w23 = jnp.concatenate([w2, w3, pad], axis=1).astype(jnp.bfloat16)
    o_ref[...] = jnp.dot(
        h, w23, preferred_element_type=jnp.float32
    ).astype(o_ref.dtype)


def _agg2_kernel(adjq_ref, sup_ref, b2_ref, b3_ref, o_ref, p_ref, *, c1, c2):
    z = jnp.dot(adjq_ref[...], sup_ref[...], preferred_element_type=jnp.float32)
    f23 = z.shape[1]
    pad = jnp.zeros((f23 - c1 - c2,), jnp.float32)
    b23 = jnp.concatenate([b2_ref[...], b3_ref[...], pad])
    z = z * (1.0 / _QSCALE) + b23[None, :]

    lane = lax.broadcasted_iota(jnp.int32, z.shape, 1)
    minus_inf = jnp.float32(-jnp.inf)

    def group_stats(mask):
        zg = jnp.where(mask, z, minus_inf)
        m = jnp.max(zg, axis=-1, keepdims=True)
        e = jnp.exp(zg - m)
        return m, e, jnp.sum(e, axis=-1, keepdims=True)

    mask1 = lane < c1
    mask2 = (lane >= c1) & (lane < c1 + c2)
    m1, e1, s1 = group_stats(mask1)
    m2, _, s2 = group_stats(mask2)
    out = jnp.where(mask1, z - m1 - jnp.log(s1), z - m2 - jnp.log(s2))
    o_ref[...] = out[:, :o_ref.shape[1]]
    prob_last = (
        jnp.sum(jnp.where(lane == c1 - 1, e1, 0.0), axis=-1, keepdims=True)
        / s1
    )
    p_ref[...] = prob_last.reshape(p_ref.shape)


def _row_slab_call(body, n_p, tm, out_shapes, out_specs, operands,
                   operand_specs):
    return pl.pallas_call(
        body,
        out_shape=out_shapes,
        grid=(n_p // tm,),
        in_specs=operand_specs,
        out_specs=out_specs,
        compiler_params=pltpu.CompilerParams(
            dimension_semantics=("parallel",),
            vmem_limit_bytes=_VMEM,
        ),
    )(*operands)


def kernel(gc1_w, gc1_b, gc2_w, gc2_b, gc3_w, gc3_b, x, adj):
    n, nfeat = x.shape
    nhid = gc1_w.shape[1]
    c1 = gc2_w.shape[1]
    c2 = gc3_w.shape[1]
    f23 = _ceil_to(c1 + c2 + 1, _LANE)

    n_p = _ceil_to(n, max(_TM1, _TM2))
    if n_p != n:
        x = jnp.pad(x, ((0, n_p - n), (0, 0)))
        adj = jnp.pad(adj, ((0, n_p - n), (0, n_p - n)))

    f_out = 16

    whole = lambda shape: pl.BlockSpec(shape, lambda i: (0,) * len(shape))
    row_slab = lambda tm, cols: pl.BlockSpec((tm, cols), lambda i: (i, 0))
    f8 = jnp.float8_e4m3fn

    sup23, adjq = _row_slab_call(
        functools.partial(_agg1_kernel, f23=f23),
        n_p, _TM1,
        (jax.ShapeDtypeStruct((n_p, f23), f8),
         jax.ShapeDtypeStruct((n_p, n_p), f8)),
        (row_slab(_TM1, f23), row_slab(_TM1, n_p)),
        (adj, x, gc1_w, gc1_b, gc2_w, gc3_w),
        [row_slab(_TM1, n_p), whole((n_p, nfeat)), whole((nfeat, nhid)),
         whole((nhid,)), whole((nhid, c1)), whole((nhid, c2))],
    )
    slab, pmat = _row_slab_call(
        functools.partial(_agg2_kernel, c1=c1, c2=c2),
        n_p, _TM2,
        (jax.ShapeDtypeStruct((n_p, f_out), jnp.float32),
         jax.ShapeDtypeStruct((n_p // _LANE, _LANE), jnp.float32)),
        (row_slab(_TM2, f_out), row_slab(_TM2 // _LANE, _LANE)),
        (adjq, sup23, gc2_b, gc3_b),
        [row_slab(_TM2, n_p), whole((n_p, f23)), whole((c1,)),
         whole((c2,))],
    )

    return slab[:n, :c1], slab[:n, c1:c1 + c2], pmat.reshape(n_p)[:n]


# speedup vs baseline: 1.2810x; 1.2059x over previous
import functools

import jax
import jax.numpy as jnp
from jax import lax
from jax.experimental import pallas as pl
from jax.experimental.pallas import tpu as pltpu

_LANE = 128
_TM1 = 512
_TM2 = 1024
_VMEM = 64 * 1024 * 1024
_QSCALE = 128.0


def _ceil_to(v, m):
    return ((v + m - 1) // m) * m


def _agg1_kernel(adj_ref, x_ref, w1_ref, b1_ref, w2t_ref, w3t_ref, o_ref,
                 adjq_ref, *, f23):
    a = adj_ref[...]
    adjq_ref[...] = (a * _QSCALE).astype(adjq_ref.dtype)
    t = jnp.dot(
        a.astype(jnp.bfloat16), x_ref[...].astype(jnp.bfloat16),
        preferred_element_type=jnp.float32,
    )
    h = jnp.dot(t, w1_ref[...], preferred_element_type=jnp.float32)
    h = jnp.maximum(h + b1_ref[...][None, :], 0.0).astype(jnp.bfloat16)
    w2t, w3t = w2t_ref[...], w3t_ref[...]
    pad = jnp.zeros(
        (f23 - w2t.shape[0] - w3t.shape[0], w2t.shape[1]), jnp.float32)
    w23t = jnp.concatenate([w2t, w3t, pad], axis=0).astype(jnp.bfloat16)
    o_ref[...] = lax.dot_general(
        h, w23t, (((1,), (1,)), ((), ())),
        preferred_element_type=jnp.float32,
    ).astype(o_ref.dtype)


def _agg2_kernel(adjq_ref, sup_ref, b2_ref, b3_ref, o1t_ref, o2t_ref, p_ref,
                 *, c1, c2):
    z = jnp.dot(adjq_ref[...], sup_ref[...], preferred_element_type=jnp.float32)
    f23 = z.shape[1]
    pad = jnp.zeros((f23 - c1 - c2,), jnp.float32)
    b23 = jnp.concatenate([b2_ref[...], b3_ref[...], pad])
    z = z * (1.0 / _QSCALE) + b23[None, :]

    lane = lax.broadcasted_iota(jnp.int32, z.shape, 1)
    minus_inf = jnp.float32(-jnp.inf)

    def group_stats(mask):
        zg = jnp.where(mask, z, minus_inf)
        m = jnp.max(zg, axis=-1, keepdims=True)
        e = jnp.exp(zg - m)
        return m, e, jnp.sum(e, axis=-1, keepdims=True)

    mask1 = lane < c1
    mask2 = (lane >= c1) & (lane < c1 + c2)
    m1, e1, s1 = group_stats(mask1)
    m2, _, s2 = group_stats(mask2)
    out = jnp.where(mask1, z - m1 - jnp.log(s1), z - m2 - jnp.log(s2))
    out_t = out[:, :c1 + c2].T
    o1t_ref[...] = out_t[:c1]
    o2t_ref[...] = out_t[c1:c1 + c2]
    prob_last = (
        jnp.sum(jnp.where(lane == c1 - 1, e1, 0.0), axis=-1, keepdims=True)
        / s1
    )
    p_ref[...] = prob_last.reshape(p_ref.shape)


def _row_slab_call(body, n_p, tm, out_shapes, out_specs, operands,
                   operand_specs):
    return pl.pallas_call(
        body,
        out_shape=out_shapes,
        grid=(n_p // tm,),
        in_specs=operand_specs,
        out_specs=out_specs,
        compiler_params=pltpu.CompilerParams(
            dimension_semantics=("parallel",),
            vmem_limit_bytes=_VMEM,
        ),
    )(*operands)


def kernel(gc1_w, gc1_b, gc2_w, gc2_b, gc3_w, gc3_b, x, adj):
    n, nfeat = x.shape
    nhid = gc1_w.shape[1]
    c1 = gc2_w.shape[1]
    c2 = gc3_w.shape[1]
    f23 = _ceil_to(c1 + c2 + 1, _LANE)

    n_p = _ceil_to(n, max(_TM1, _TM2))
    if n_p != n:
        x = jnp.pad(x, ((0, n_p - n), (0, 0)))
        adj = jnp.pad(adj, ((0, n_p - n), (0, n_p - n)))

    whole = lambda shape: pl.BlockSpec(shape, lambda i: (0,) * len(shape))
    row_slab = lambda tm, cols: pl.BlockSpec((tm, cols), lambda i: (i, 0))
    col_slab = lambda rows, tm: pl.BlockSpec((rows, tm), lambda i: (0, i))
    f8 = jnp.float8_e4m3fn

    sup23, adjq = _row_slab_call(
        functools.partial(_agg1_kernel, f23=f23),
        n_p, _TM1,
        (jax.ShapeDtypeStruct((n_p, f23), f8),
         jax.ShapeDtypeStruct((n_p, n_p), f8)),
        (row_slab(_TM1, f23), row_slab(_TM1, n_p)),
        (adj, x, gc1_w, gc1_b, gc2_w.T, gc3_w.T),
        [row_slab(_TM1, n_p), whole((n_p, nfeat)), whole((nfeat, nhid)),
         whole((nhid,)), whole((c1, nhid)), whole((c2, nhid))],
    )
    ls1t, ls2t, pmat = _row_slab_call(
        functools.partial(_agg2_kernel, c1=c1, c2=c2),
        n_p, _TM2,
        (jax.ShapeDtypeStruct((c1, n_p), jnp.float32),
         jax.ShapeDtypeStruct((c2, n_p), jnp.float32),
         jax.ShapeDtypeStruct((n_p // _LANE, _LANE), jnp.float32)),
        (col_slab(c1, _TM2), col_slab(c2, _TM2),
         row_slab(_TM2 // _LANE, _LANE)),
        (adjq, sup23, gc2_b, gc3_b),
        [row_slab(_TM2, n_p), whole((n_p, f23)), whole((c1,)),
         whole((c2,))],
    )

    return ls1t.T[:n], ls2t.T[:n], pmat.reshape(n_p)[:n]
